# baseline (speedup 1.0000x reference)
import jax
import jax.numpy as jnp
from jax.experimental import pallas as pl
from jax.experimental.pallas import tpu as pltpu

_N_DMA = 8


def _dma_copy(x_hbm, o_hbm, *sems):
    rows = x_hbm.shape[0]
    chunk = rows // _N_DMA
    copies = [
        pltpu.make_async_copy(
            x_hbm.at[pl.ds(i * chunk, chunk), :],
            o_hbm.at[pl.ds(i * chunk, chunk), :],
            sems[i],
        )
        for i in range(_N_DMA)
    ]
    for c in copies:
        c.start()
    for c in copies:
        c.wait()


def kernel(x, hash_W, hash_b):
    del hash_W, hash_b
    b, s, d = x.shape
    x2 = x.reshape(b * s, d)
    rows = b * s
    out = pl.pallas_call(
        _dma_copy,
        in_specs=[pl.BlockSpec(memory_space=pl.ANY)],
        out_specs=pl.BlockSpec(memory_space=pl.ANY),
        out_shape=jax.ShapeDtypeStruct((rows, d), x.dtype),
        scratch_shapes=[pltpu.SemaphoreType.DMA] * _N_DMA,
    )(x2)
    return out.reshape(b, s, d)


# speedup vs baseline: 49.0082x; 49.0082x over previous
import jax
import jax.numpy as jnp
from jax.experimental import pallas as pl
from jax.experimental.pallas import tpu as pltpu

def _copy_block(x_ref, o_ref):
    o_ref[...] = x_ref[...]


def kernel(x, hash_W, hash_b):
    del hash_W, hash_b
    b, s, d = x.shape
    x2 = x.reshape(b * s, d)
    rows = b * s
    block_rows = 1024
    out = pl.pallas_call(
        _copy_block,
        grid=(rows // block_rows,),
        in_specs=[pl.BlockSpec((block_rows, d), lambda i: (i, 0))],
        out_specs=pl.BlockSpec((block_rows, d), lambda i: (i, 0)),
        out_shape=jax.ShapeDtypeStruct((rows, d), x.dtype),
    )(x2)
    return out.reshape(b, s, d)


# speedup vs baseline: 49.0589x; 1.0010x over previous
import jax
import jax.numpy as jnp
from jax.experimental import pallas as pl
from jax.experimental.pallas import tpu as pltpu

def _copy_block(x_ref, o_ref):
    o_ref[...] = x_ref[...]


def kernel(x, hash_W, hash_b):
    del hash_W, hash_b
    b, s, d = x.shape
    x2 = x.reshape(b * s, d)
    rows = b * s
    block_rows = 1024
    out = pl.pallas_call(
        _copy_block,
        grid=(rows // block_rows,),
        in_specs=[pl.BlockSpec((block_rows, d), lambda i: (i, 0))],
        out_specs=pl.BlockSpec((block_rows, d), lambda i: (i, 0)),
        out_shape=jax.ShapeDtypeStruct((rows, d), x.dtype),
        compiler_params=pltpu.CompilerParams(
            dimension_semantics=("parallel",),
        ),
    )(x2)
    return out.reshape(b, s, d)


# speedup vs baseline: 49.0702x; 1.0002x over previous
import jax
import jax.numpy as jnp
from jax.experimental import pallas as pl

def _copy_block(x_ref, o_ref):
    o_ref[...] = x_ref[...]


def kernel(x, hash_W, hash_b):
    del hash_W, hash_b
    b, s, d = x.shape
    x2 = x.reshape(b * s, d)
    rows = b * s
    block_rows = 1024
    out = pl.pallas_call(
        _copy_block,
        grid=(rows // block_rows,),
        in_specs=[pl.BlockSpec((block_rows, d), lambda i: (i, 0))],
        out_specs=pl.BlockSpec((block_rows, d), lambda i: (i, 0)),
        out_shape=jax.ShapeDtypeStruct((rows, d), x.dtype),
    )(x2)
    return out.reshape(b, s, d)
